# baseline (speedup 1.0000x reference)
import functools

import jax
import jax.numpy as jnp
from jax import lax
from jax.experimental import pallas as pl
from jax.experimental.pallas import tpu as pltpu
from jax.experimental.pallas import tpu_sc as plsc

NC = 2
NS = 16
LANES = 16
DB = 128
CHUNK = 40
SBS = 40


def _proj_body(x_ref, w_ref, b_ref, k0, k1, qv0, qv1):
    bm = x_ref.shape[0]
    acc = jnp.dot(x_ref[...], w_ref[...], preferred_element_type=jnp.float32)
    acc = acc + b_ref[...][None, :]
    k0[...] = acc[:, 0 * DB:1 * DB]
    k1[...] = acc[:, 1 * DB:2 * DB]
    qv0[...] = jnp.stack([acc[:, 2 * DB:3 * DB], acc[:, 4 * DB:5 * DB]],
                         axis=1).reshape(2 * bm, DB)
    qv1[...] = jnp.stack([acc[:, 3 * DB:4 * DB], acc[:, 5 * DB:6 * DB]],
                         axis=1).reshape(2 * bm, DB)


def _proj(x, w, b, bm):
    n, d = x.shape
    grid = n // bm
    outs = [jax.ShapeDtypeStruct((n, DB), jnp.float32)] * 2 + \
           [jax.ShapeDtypeStruct((2 * n, DB), jnp.float32)] * 2
    out_specs = [pl.BlockSpec((bm, DB), lambda i: (i, 0))] * 2 + \
                [pl.BlockSpec((2 * bm, DB), lambda i: (i, 0))] * 2
    return pl.pallas_call(
        _proj_body,
        grid=(grid,),
        in_specs=[
            pl.BlockSpec((bm, d), lambda i: (i, 0)),
            pl.BlockSpec(w.shape, lambda i: (0, 0)),
            pl.BlockSpec(b.shape, lambda i: (0,)),
        ],
        out_specs=out_specs,
        out_shape=outs,
    )(x, w, b)


def _eproj_body(e_ref, w_ref, b_ref, e0, e1):
    acc = jnp.dot(e_ref[...], w_ref[...], preferred_element_type=jnp.float32)
    acc = acc + b_ref[...][None, :]
    e0[...] = acc[:, :DB]
    e1[...] = acc[:, DB:]


def _eproj(xe, w, b, bm):
    m, de = xe.shape
    grid = m // bm
    outs = [jax.ShapeDtypeStruct((m, DB), jnp.float32)] * 2
    out_specs = [pl.BlockSpec((bm, DB), lambda i: (i, 0))] * 2
    return pl.pallas_call(
        _eproj_body,
        grid=(grid,),
        in_specs=[
            pl.BlockSpec((bm, de), lambda i: (i, 0)),
            pl.BlockSpec(w.shape, lambda i: (0, 0)),
            pl.BlockSpec(b.shape, lambda i: (0,)),
        ],
        out_specs=out_specs,
        out_shape=outs,
    )(xe, w, b)


def _combine_body(p_ref, x_ref, w_ref, b_ref, o_ref):
    agg = p_ref[0, 0] + p_ref[1, 0]
    acc = jnp.dot(x_ref[...], w_ref[...], preferred_element_type=jnp.float32)
    o_ref[...] = jnp.maximum(acc + agg + b_ref[...][None, :], 0.0)


def _combine(part, x, w, b, bm):
    n, d = x.shape
    grid = (n // bm, d // DB)
    return pl.pallas_call(
        _combine_body,
        grid=grid,
        in_specs=[
            pl.BlockSpec((NC, 1, bm, DB), lambda i, j: (0, j, i, 0)),
            pl.BlockSpec((bm, d), lambda i, j: (i, 0)),
            pl.BlockSpec((d, DB), lambda i, j: (0, j)),
            pl.BlockSpec((DB,), lambda i, j: (j,)),
        ],
        out_specs=pl.BlockSpec((bm, DB), lambda i, j: (i, j)),
        out_shape=jax.ShapeDtypeStruct((n, d), jnp.float32),
    )(part, x, w, b)


def _edge_body(n_nodes, n_edges,
               k0, k1, qv0, qv1, e0, e1, comb,
               out, combb, kba, kbb, qvba, qvbb, eba, acc,
               sga, sgb, ssa, ssb):
    c = lax.axis_index("c")
    s = lax.axis_index("s")
    wid = s * NC + c

    e_per_tile = n_edges // (NC * NS)
    n_chunks = e_per_tile // CHUNK
    cpc = 3 * CHUNK
    rows_main = -(-n_nodes // NS) // 8 * 8
    rows_last = n_nodes - (NS - 1) * rows_main
    row_start = s * rows_main
    is_last = s == NS - 1

    def do_pass(kt, qvt, et):
        edge0 = wid * e_per_tile

        def d_idx(ci):
            return combb.at[pl.ds(ci * cpc, CHUNK)]

        def qv_idx(ci):
            return combb.at[pl.ds(ci * cpc + CHUNK, 2 * CHUNK)]

        def fire_kqv(kb, qvb, ci, sem):
            pltpu.async_copy(kt.at[d_idx(ci)], kb, sem)
            pltpu.async_copy(qvt.at[qv_idx(ci)], qvb, sem)

        def fire_e(ci, sem):
            pltpu.async_copy(et.at[pl.ds(edge0 + ci * CHUNK, CHUNK)],
                             eba, sem)

        def wait_g(kb, qvb, sem):
            pltpu.make_async_copy(kt.at[d_idx(0)], kb, sem).wait()
            pltpu.make_async_copy(qvt.at[qv_idx(0)], qvb, sem).wait()
            pltpu.make_async_copy(et.at[pl.ds(edge0, CHUNK)], eba, sem).wait()

        def wait_s(kb, sem):
            pltpu.make_async_copy(kb, acc.at[d_idx(0)], sem).wait()

        def compute(kb, qvb):
            def _row(r, _):
                for cb in range(DB // LANES):
                    sl = pl.ds(cb * LANES, LANES)
                    z = kb[r, sl] + qvb[2 * r, sl] + eba[r, sl]
                    gate = 1.0 / (1.0 + jnp.exp(-z))
                    kb[r, sl] = gate * qvb[2 * r + 1, sl]
                return 0
            lax.fori_loop(0, CHUNK, _row, 0)

        npairs = n_chunks // 2

        fire_kqv(kba, qvba, 0, sga)

        def pair(j, _):
            c0 = 2 * j
            c1 = c0 + 1
            fire_e(c0, sga)

            @pl.when(j > 0)
            def _():
                wait_s(kbb, ssb)
            fire_kqv(kbb, qvbb, c1, sgb)
            wait_g(kba, qvba, sga)
            compute(kba, qvba)
            pltpu.async_copy(kba, acc.at[d_idx(c0)], ssa, add=True)
            fire_e(c1, sgb)
            wait_g(kbb, qvbb, sgb)
            compute(kbb, qvbb)
            pltpu.async_copy(kbb, acc.at[d_idx(c1)], ssb, add=True)
            wait_s(kba, ssa)
            fire_kqv(kba, qvba, c0 + 2, sga)
            return 0

        lax.fori_loop(0, npairs, pair, 0)
        fire_e(n_chunks - 1, sga)
        wait_g(kba, qvba, sga)
        compute(kba, qvba)
        pltpu.sync_copy(kba, acc.at[d_idx(n_chunks - 1)], add=True)
        wait_s(kbb, ssb)

    pltpu.sync_copy(comb.at[pl.ds(wid * n_chunks * cpc, n_chunks * cpc)],
                    combb)

    for db, (kt, qvt, et) in enumerate(((k0, qv0, e0), (k1, qv1, e1))):
        for r in range(8):
            for cb in range(DB // LANES):
                eba[r, pl.ds(cb * LANES, LANES)] = jnp.zeros((LANES,),
                                                             jnp.float32)
        n_clear = jnp.where(is_last, rows_last // 8, rows_main // 8)

        def _clear(i, _):
            pltpu.sync_copy(eba.at[pl.ds(0, 8)],
                            acc.at[pl.ds(row_start + i * 8, 8)])
            return 0
        lax.fori_loop(0, n_clear, _clear, 0)
        plsc.subcore_barrier()

        do_pass(kt, qvt, et)
        plsc.subcore_barrier()

        @pl.when(jnp.logical_not(is_last))
        def _():
            pltpu.sync_copy(acc.at[pl.ds(row_start, rows_main)],
                            out.at[c, db, pl.ds(row_start, rows_main)])

        @pl.when(is_last)
        def _():
            pltpu.sync_copy(acc.at[pl.ds(row_start, rows_last)],
                            out.at[c, db, pl.ds(row_start, rows_last)])
        plsc.subcore_barrier()


def _edge_sc(k0, k1, qv0, qv1, e0, e1, comb):
    n_nodes = k0.shape[0]
    n_edges = e0.shape[0]
    e_per_tile = n_edges // (NC * NS)
    comb_per_tile = (e_per_tile // CHUNK) * 3 * CHUNK

    mesh = plsc.VectorSubcoreMesh(core_axis_name="c", subcore_axis_name="s")
    body = functools.partial(_edge_body, n_nodes, n_edges)
    return pl.kernel(
        body,
        out_type=jax.ShapeDtypeStruct((NC, 2, n_nodes, DB), jnp.float32),
        mesh=mesh,
        scratch_types=[
            pltpu.VMEM((comb_per_tile,), jnp.int32),
            pltpu.VMEM((CHUNK, DB), jnp.float32),
            pltpu.VMEM((CHUNK, DB), jnp.float32),
            pltpu.VMEM((2 * CHUNK, DB), jnp.float32),
            pltpu.VMEM((2 * CHUNK, DB), jnp.float32),
            pltpu.VMEM((CHUNK, DB), jnp.float32),
            pltpu.VMEM_SHARED((n_nodes, DB), jnp.float32),
            pltpu.SemaphoreType.DMA,
            pltpu.SemaphoreType.DMA,
            pltpu.SemaphoreType.DMA,
            pltpu.SemaphoreType.DMA,
        ],
    )(k0, k1, qv0, qv1, e0, e1, comb)


def _layer(x, comb, eproj01, p):
    wcat = jnp.concatenate([p['Wk'], p['Wq'], p['Wv']], axis=1)
    bcat = jnp.concatenate([p['bk'], p['bq'], p['bv']])
    k0, k1, qv0, qv1 = _proj(x, wcat, bcat, bm=400)
    e0, e1 = eproj01
    part = _edge_sc(k0, k1, qv0, qv1, e0, e1, comb)
    return _combine(part, x, p['Ws'], p['b'], bm=400)


def kernel(x, edge_index, x_edge, params):
    n_edges = edge_index.shape[1]
    src = edge_index[0]
    dst = edge_index[1]
    dst_c = dst.reshape(n_edges // CHUNK, CHUNK)
    src_qv = jnp.stack([2 * src, 2 * src + 1],
                       axis=-1).reshape(n_edges // CHUNK, 2 * CHUNK)
    comb = jnp.concatenate([dst_c, src_qv], axis=1).reshape(-1)
    ep1 = _eproj(x_edge, params['conv1']['We'], params['conv1']['be'], bm=1000)
    ep2 = _eproj(x_edge, params['conv2']['We'], params['conv2']['be'], bm=1000)
    h = _layer(x, comb, ep1, params['conv1'])
    h = _layer(h, comb, ep2, params['conv2'])
    return h


# speedup vs baseline: 2.9311x; 2.9311x over previous
import functools

import jax
import jax.numpy as jnp
from jax import lax
from jax.experimental import pallas as pl
from jax.experimental.pallas import tpu as pltpu
from jax.experimental.pallas import tpu_sc as plsc

NC = 2
NS = 16
LANES = 16
DB = 128
CHUNK = 40
SBS = 40


def _proj_body(x_ref, w_ref, b_ref, k0, k1, q0, q1, v0, v1):
    acc = jnp.dot(x_ref[...], w_ref[...], preferred_element_type=jnp.float32)
    acc = acc + b_ref[...][None, :]
    k0[...] = acc[:, 0 * DB:1 * DB]
    k1[...] = acc[:, 1 * DB:2 * DB]
    q0[...] = acc[:, 2 * DB:3 * DB]
    q1[...] = acc[:, 3 * DB:4 * DB]
    v0[...] = acc[:, 4 * DB:5 * DB]
    v1[...] = acc[:, 5 * DB:6 * DB]


def _proj(x, w, b, bm):
    n, d = x.shape
    grid = n // bm
    outs = [jax.ShapeDtypeStruct((n, DB), jnp.float32)] * 6
    out_specs = [pl.BlockSpec((bm, DB), lambda i: (i, 0))] * 6
    return pl.pallas_call(
        _proj_body,
        grid=(grid,),
        in_specs=[
            pl.BlockSpec((bm, d), lambda i: (i, 0)),
            pl.BlockSpec(w.shape, lambda i: (0, 0)),
            pl.BlockSpec(b.shape, lambda i: (0,)),
        ],
        out_specs=out_specs,
        out_shape=outs,
    )(x, w, b)


def _eproj_body(e_ref, w_ref, b_ref, e0, e1):
    acc = jnp.dot(e_ref[...], w_ref[...], preferred_element_type=jnp.float32)
    acc = acc + b_ref[...][None, :]
    e0[...] = acc[:, :DB]
    e1[...] = acc[:, DB:]


def _eproj(xe, w, b, bm):
    m, de = xe.shape
    grid = m // bm
    outs = [jax.ShapeDtypeStruct((m, DB), jnp.float32)] * 2
    out_specs = [pl.BlockSpec((bm, DB), lambda i: (i, 0))] * 2
    return pl.pallas_call(
        _eproj_body,
        grid=(grid,),
        in_specs=[
            pl.BlockSpec((bm, de), lambda i: (i, 0)),
            pl.BlockSpec(w.shape, lambda i: (0, 0)),
            pl.BlockSpec(b.shape, lambda i: (0,)),
        ],
        out_specs=out_specs,
        out_shape=outs,
    )(xe, w, b)


def _combine_body(p_ref, x_ref, w_ref, b_ref, o_ref):
    agg = p_ref[0, 0] + p_ref[1, 0]
    acc = jnp.dot(x_ref[...], w_ref[...], preferred_element_type=jnp.float32)
    o_ref[...] = jnp.maximum(acc + agg + b_ref[...][None, :], 0.0)


def _combine(part, x, w, b, bm):
    n, d = x.shape
    grid = (n // bm, d // DB)
    return pl.pallas_call(
        _combine_body,
        grid=grid,
        in_specs=[
            pl.BlockSpec((NC, 1, bm, DB), lambda i, j: (0, j, i, 0)),
            pl.BlockSpec((bm, d), lambda i, j: (i, 0)),
            pl.BlockSpec((d, DB), lambda i, j: (0, j)),
            pl.BlockSpec((DB,), lambda i, j: (j,)),
        ],
        out_specs=pl.BlockSpec((bm, DB), lambda i, j: (i, j)),
        out_shape=jax.ShapeDtypeStruct((n, d), jnp.float32),
    )(part, x, w, b)


def _edge_body(n_nodes, n_edges,
               k0, k1, q0, q1, v0, v1, e0, e1, src_r, dst_r,
               out, srcb, dstb, kba, kbb, qba, qbb, vba, vbb, eba, zb, acc,
               sga, sgb, ssa, ssb):
    c = lax.axis_index("c")
    s = lax.axis_index("s")
    wid = s * NC + c

    e_per_tile = n_edges // (NC * NS)
    n_chunks = e_per_tile // CHUNK
    rows_main = -(-n_nodes // NS) // 8 * 8
    rows_last = n_nodes - (NS - 1) * rows_main
    row_start = s * rows_main
    is_last = s == NS - 1

    for r in range(zb.shape[0]):
        for cb in range(DB // LANES):
            zb[r, pl.ds(cb * LANES, LANES)] = jnp.zeros((LANES,), jnp.float32)

    def do_pass(kt, qt, vt, et):
        pltpu.sync_copy(dst_r.at[pl.ds(wid * e_per_tile, e_per_tile)], dstb)
        pltpu.sync_copy(src_r.at[pl.ds(wid * e_per_tile, e_per_tile)], srcb)
        edge0 = wid * e_per_tile

        def d_idx(ci):
            return dstb.at[pl.ds(ci * CHUNK, CHUNK)]

        def s_idx(ci):
            return srcb.at[pl.ds(ci * CHUNK, CHUNK)]

        def fire_kqv(kb, qb, vb, ci, sem):
            pltpu.async_copy(kt.at[d_idx(ci)], kb, sem)
            pltpu.async_copy(qt.at[s_idx(ci)], qb, sem)
            pltpu.async_copy(vt.at[s_idx(ci)], vb, sem)

        def fire_e(ci, sem):
            pltpu.async_copy(et.at[pl.ds(edge0 + ci * CHUNK, CHUNK)],
                             eba, sem)

        def wait_g(kb, qb, vb, sem):
            pltpu.make_async_copy(kt.at[d_idx(0)], kb, sem).wait()
            pltpu.make_async_copy(qt.at[s_idx(0)], qb, sem).wait()
            pltpu.make_async_copy(vt.at[s_idx(0)], vb, sem).wait()
            pltpu.make_async_copy(et.at[pl.ds(edge0, CHUNK)], eba, sem).wait()

        def wait_s(kb, sem):
            pltpu.make_async_copy(kb, acc.at[d_idx(0)], sem).wait()

        def compute(kb, qb, vb):
            def _row(r, _):
                for cb in range(DB // LANES):
                    sl = pl.ds(cb * LANES, LANES)
                    z = kb[r, sl] + qb[r, sl] + eba[r, sl]
                    gate = 1.0 / (1.0 + jnp.exp(-z))
                    kb[r, sl] = gate * vb[r, sl]
                return 0
            lax.fori_loop(0, CHUNK, _row, 0)

        npairs = n_chunks // 2

        fire_kqv(kba, qba, vba, 0, sga)

        def pair(j, _):
            c0 = 2 * j
            c1 = c0 + 1
            fire_e(c0, sga)

            @pl.when(j > 0)
            def _():
                wait_s(kbb, ssb)
            fire_kqv(kbb, qbb, vbb, c1, sgb)
            wait_g(kba, qba, vba, sga)
            compute(kba, qba, vba)
            pltpu.async_copy(kba, acc.at[d_idx(c0)], ssa, add=True)
            fire_e(c1, sgb)
            wait_g(kbb, qbb, vbb, sgb)
            compute(kbb, qbb, vbb)
            pltpu.async_copy(kbb, acc.at[d_idx(c1)], ssb, add=True)
            wait_s(kba, ssa)
            fire_kqv(kba, qba, vba, c0 + 2, sga)
            return 0

        lax.fori_loop(0, npairs, pair, 0)
        fire_e(n_chunks - 1, sga)
        wait_g(kba, qba, vba, sga)
        compute(kba, qba, vba)
        pltpu.sync_copy(kba, acc.at[d_idx(n_chunks - 1)], add=True)
        wait_s(kbb, ssb)

    for db, (kt, qt, vt, et) in enumerate(((k0, q0, v0, e0),
                                           (k1, q1, v1, e1))):
        n_clear = jnp.where(is_last, rows_last // 8, rows_main // 8)

        def _clear(i, _):
            pltpu.sync_copy(zb, acc.at[pl.ds(row_start + i * 8, 8)])
            return 0
        lax.fori_loop(0, n_clear, _clear, 0)
        plsc.subcore_barrier()

        do_pass(kt, qt, vt, et)
        plsc.subcore_barrier()

        @pl.when(jnp.logical_not(is_last))
        def _():
            pltpu.sync_copy(acc.at[pl.ds(row_start, rows_main)],
                            out.at[c, db, pl.ds(row_start, rows_main)])

        @pl.when(is_last)
        def _():
            pltpu.sync_copy(acc.at[pl.ds(row_start, rows_last)],
                            out.at[c, db, pl.ds(row_start, rows_last)])
        plsc.subcore_barrier()


def _edge_sc(k0, k1, q0, q1, v0, v1, e0, e1, src_r, dst_r):
    n_nodes = k0.shape[0]
    n_edges = e0.shape[0]
    e_per_tile = n_edges // (NC * NS)

    mesh = plsc.VectorSubcoreMesh(core_axis_name="c", subcore_axis_name="s")
    body = functools.partial(_edge_body, n_nodes, n_edges)
    return pl.kernel(
        body,
        out_type=jax.ShapeDtypeStruct((NC, 2, n_nodes, DB), jnp.float32),
        mesh=mesh,
        scratch_types=[
            pltpu.VMEM((e_per_tile,), jnp.int32),
            pltpu.VMEM((e_per_tile,), jnp.int32),
            pltpu.VMEM((CHUNK, DB), jnp.float32),
            pltpu.VMEM((CHUNK, DB), jnp.float32),
            pltpu.VMEM((CHUNK, DB), jnp.float32),
            pltpu.VMEM((CHUNK, DB), jnp.float32),
            pltpu.VMEM((CHUNK, DB), jnp.float32),
            pltpu.VMEM((CHUNK, DB), jnp.float32),
            pltpu.VMEM((CHUNK, DB), jnp.float32),
            pltpu.VMEM((8, DB), jnp.float32),
            pltpu.VMEM_SHARED((n_nodes, DB), jnp.float32),
            pltpu.SemaphoreType.DMA,
            pltpu.SemaphoreType.DMA,
            pltpu.SemaphoreType.DMA,
            pltpu.SemaphoreType.DMA,
        ],
    )(k0, k1, q0, q1, v0, v1, e0, e1, src_r, dst_r)


def _layer(x, src_r, dst_r, eproj01, p):
    wcat = jnp.concatenate([p['Wk'], p['Wq'], p['Wv']], axis=1)
    bcat = jnp.concatenate([p['bk'], p['bq'], p['bv']])
    k0, k1, q0, q1, v0, v1 = _proj(x, wcat, bcat, bm=400)
    e0, e1 = eproj01
    part = _edge_sc(k0, k1, q0, q1, v0, v1, e0, e1, src_r, dst_r)
    return _combine(part, x, p['Ws'], p['b'], bm=400)


def kernel(x, edge_index, x_edge, params):
    src_r = edge_index[0]
    dst_r = edge_index[1]
    ep1 = _eproj(x_edge, params['conv1']['We'], params['conv1']['be'], bm=1000)
    ep2 = _eproj(x_edge, params['conv2']['We'], params['conv2']['be'], bm=1000)
    h = _layer(x, src_r, dst_r, ep1, params['conv1'])
    h = _layer(h, src_r, dst_r, ep2, params['conv2'])
    return h


# speedup vs baseline: 2.9362x; 1.0018x over previous
import functools

import jax
import jax.numpy as jnp
from jax import lax
from jax.experimental import pallas as pl
from jax.experimental.pallas import tpu as pltpu
from jax.experimental.pallas import tpu_sc as plsc

NC = 2
NS = 16
LANES = 16
DB = 128
CHUNK = 40


def _proj_body(x_ref, w_ref, b_ref, k0, k1, q0, q1, v0, v1):
    acc = jnp.dot(x_ref[...], w_ref[...], preferred_element_type=jnp.float32)
    acc = acc + b_ref[...][None, :]
    k0[...] = acc[:, 0 * DB:1 * DB]
    k1[...] = acc[:, 1 * DB:2 * DB]
    q0[...] = acc[:, 2 * DB:3 * DB]
    q1[...] = acc[:, 3 * DB:4 * DB]
    v0[...] = acc[:, 4 * DB:5 * DB]
    v1[...] = acc[:, 5 * DB:6 * DB]


def _proj(x, w, b, bm):
    n, d = x.shape
    grid = n // bm
    outs = [jax.ShapeDtypeStruct((n, DB), jnp.float32)] * 6
    out_specs = [pl.BlockSpec((bm, DB), lambda i: (i, 0))] * 6
    return pl.pallas_call(
        _proj_body,
        grid=(grid,),
        in_specs=[
            pl.BlockSpec((bm, d), lambda i: (i, 0)),
            pl.BlockSpec(w.shape, lambda i: (0, 0)),
            pl.BlockSpec(b.shape, lambda i: (0,)),
        ],
        out_specs=out_specs,
        out_shape=outs,
    )(x, w, b)


def _eproj_body(e_ref, w_ref, b_ref, e0, e1):
    acc = jnp.dot(e_ref[...], w_ref[...], preferred_element_type=jnp.float32)
    acc = acc + b_ref[...][None, :]
    e0[...] = acc[:, :DB]
    e1[...] = acc[:, DB:]


def _eproj(xe, w, b, bm):
    m, de = xe.shape
    grid = m // bm
    outs = [jax.ShapeDtypeStruct((m, DB), jnp.float32)] * 2
    out_specs = [pl.BlockSpec((bm, DB), lambda i: (i, 0))] * 2
    return pl.pallas_call(
        _eproj_body,
        grid=(grid,),
        in_specs=[
            pl.BlockSpec((bm, de), lambda i: (i, 0)),
            pl.BlockSpec(w.shape, lambda i: (0, 0)),
            pl.BlockSpec(b.shape, lambda i: (0,)),
        ],
        out_specs=out_specs,
        out_shape=outs,
    )(xe, w, b)


def _combine_body(p_ref, x_ref, w_ref, b_ref, o_ref):
    agg = p_ref[0, 0] + p_ref[1, 0]
    acc = jnp.dot(x_ref[...], w_ref[...], preferred_element_type=jnp.float32)
    o_ref[...] = jnp.maximum(acc + agg + b_ref[...][None, :], 0.0)


def _combine(part, x, w, b, bm):
    n, d = x.shape
    grid = (n // bm, d // DB)
    return pl.pallas_call(
        _combine_body,
        grid=grid,
        in_specs=[
            pl.BlockSpec((NC, 1, bm, DB), lambda i, j: (0, j, i, 0)),
            pl.BlockSpec((bm, d), lambda i, j: (i, 0)),
            pl.BlockSpec((d, DB), lambda i, j: (0, j)),
            pl.BlockSpec((DB,), lambda i, j: (j,)),
        ],
        out_specs=pl.BlockSpec((bm, DB), lambda i, j: (i, j)),
        out_shape=jax.ShapeDtypeStruct((n, d), jnp.float32),
    )(part, x, w, b)


def _edge_body(n_nodes, n_edges,
               k0, k1, q0, q1, v0, v1, e0, e1, src_r, dst_r,
               out, srcb, dstb, kba, kbb, qba, qbb, vba, vbb, eba, zb, acc,
               sga, sgb, ssa, ssb):
    c = lax.axis_index("c")
    s = lax.axis_index("s")
    wid = s * NC + c

    e_per_tile = n_edges // (NC * NS)
    n_chunks = e_per_tile // CHUNK
    rows_main = -(-n_nodes // NS) // 8 * 8
    rows_last = n_nodes - (NS - 1) * rows_main
    row_start = s * rows_main
    is_last = s == NS - 1

    for r in range(zb.shape[0]):
        for cb in range(DB // LANES):
            zb[r, pl.ds(cb * LANES, LANES)] = jnp.zeros((LANES,), jnp.float32)

    def do_pass(kt, qt, vt, et):
        pltpu.sync_copy(dst_r.at[pl.ds(wid * e_per_tile, e_per_tile)], dstb)
        pltpu.sync_copy(src_r.at[pl.ds(wid * e_per_tile, e_per_tile)], srcb)
        edge0 = wid * e_per_tile

        def d_idx(ci):
            return dstb.at[pl.ds(ci * CHUNK, CHUNK)]

        def s_idx(ci):
            return srcb.at[pl.ds(ci * CHUNK, CHUNK)]

        def fire_kqv(kb, qb, vb, ci, sem):
            pltpu.async_copy(kt.at[d_idx(ci)], kb, sem)
            pltpu.async_copy(qt.at[s_idx(ci)], qb, sem)
            pltpu.async_copy(vt.at[s_idx(ci)], vb, sem)

        def fire_e(ci, sem):
            pltpu.async_copy(et.at[pl.ds(edge0 + ci * CHUNK, CHUNK)],
                             eba, sem)

        def wait_g(kb, qb, vb, sem):
            pltpu.make_async_copy(kt.at[d_idx(0)], kb, sem).wait()
            pltpu.make_async_copy(qt.at[s_idx(0)], qb, sem).wait()
            pltpu.make_async_copy(vt.at[s_idx(0)], vb, sem).wait()
            pltpu.make_async_copy(et.at[pl.ds(edge0, CHUNK)], eba, sem).wait()

        def wait_s(kb, sem):
            pltpu.make_async_copy(kb, acc.at[d_idx(0)], sem).wait()

        def compute(kb, qb, vb):
            def _row(r, _):
                for cb in range(DB // LANES):
                    sl = pl.ds(cb * LANES, LANES)
                    z = kb[r, sl] + qb[r, sl] + eba[r, sl]
                    gate = 1.0 / (1.0 + jnp.exp(-z))
                    kb[r, sl] = gate * vb[r, sl]
                return 0
            lax.fori_loop(0, CHUNK, _row, 0)

        npairs = n_chunks // 2

        fire_kqv(kba, qba, vba, 0, sga)

        def pair(j, _):
            c0 = 2 * j
            c1 = c0 + 1
            fire_e(c0, sga)

            @pl.when(j > 0)
            def _():
                wait_s(kbb, ssb)
            fire_kqv(kbb, qbb, vbb, c1, sgb)
            wait_g(kba, qba, vba, sga)
            compute(kba, qba, vba)
            pltpu.async_copy(kba, acc.at[d_idx(c0)], ssa, add=True)
            fire_e(c1, sgb)
            wait_g(kbb, qbb, vbb, sgb)
            compute(kbb, qbb, vbb)
            pltpu.async_copy(kbb, acc.at[d_idx(c1)], ssb, add=True)
            wait_s(kba, ssa)
            fire_kqv(kba, qba, vba, c0 + 2, sga)
            return 0

        lax.fori_loop(0, npairs, pair, 0)
        fire_e(n_chunks - 1, sga)
        wait_g(kba, qba, vba, sga)
        compute(kba, qba, vba)
        pltpu.sync_copy(kba, acc.at[d_idx(n_chunks - 1)], add=True)
        wait_s(kbb, ssb)

    for db, (kt, qt, vt, et) in enumerate(((k0, q0, v0, e0),
                                           (k1, q1, v1, e1))):
        n_clear = jnp.where(is_last, rows_last // 8, rows_main // 8)

        def _clear(i, _):
            pltpu.sync_copy(zb, acc.at[pl.ds(row_start + i * 8, 8)])
            return 0
        lax.fori_loop(0, n_clear, _clear, 0)
        plsc.subcore_barrier()

        do_pass(kt, qt, vt, et)
        plsc.subcore_barrier()

        @pl.when(jnp.logical_not(is_last))
        def _():
            pltpu.sync_copy(acc.at[pl.ds(row_start, rows_main)],
                            out.at[c, db, pl.ds(row_start, rows_main)])

        @pl.when(is_last)
        def _():
            pltpu.sync_copy(acc.at[pl.ds(row_start, rows_last)],
                            out.at[c, db, pl.ds(row_start, rows_last)])
        plsc.subcore_barrier()


def _edge_sc(k0, k1, q0, q1, v0, v1, e0, e1, src_r, dst_r):
    n_nodes = k0.shape[0]
    n_edges = e0.shape[0]
    e_per_tile = n_edges // (NC * NS)

    mesh = plsc.VectorSubcoreMesh(core_axis_name="c", subcore_axis_name="s")
    body = functools.partial(_edge_body, n_nodes, n_edges)
    return pl.kernel(
        body,
        out_type=jax.ShapeDtypeStruct((NC, 2, n_nodes, DB), jnp.float32),
        mesh=mesh,
        scratch_types=[
            pltpu.VMEM((e_per_tile,), jnp.int32),
            pltpu.VMEM((e_per_tile,), jnp.int32),
            pltpu.VMEM((CHUNK, DB), jnp.float32),
            pltpu.VMEM((CHUNK, DB), jnp.float32),
            pltpu.VMEM((CHUNK, DB), jnp.float32),
            pltpu.VMEM((CHUNK, DB), jnp.float32),
            pltpu.VMEM((CHUNK, DB), jnp.float32),
            pltpu.VMEM((CHUNK, DB), jnp.float32),
            pltpu.VMEM((CHUNK, DB), jnp.float32),
            pltpu.VMEM((8, DB), jnp.float32),
            pltpu.VMEM_SHARED((n_nodes, DB), jnp.float32),
            pltpu.SemaphoreType.DMA,
            pltpu.SemaphoreType.DMA,
            pltpu.SemaphoreType.DMA,
            pltpu.SemaphoreType.DMA,
        ],
    )(k0, k1, q0, q1, v0, v1, e0, e1, src_r, dst_r)


def _layer(x, src_r, dst_r, eproj01, p):
    wcat = jnp.concatenate([p['Wk'], p['Wq'], p['Wv']], axis=1)
    bcat = jnp.concatenate([p['bk'], p['bq'], p['bv']])
    k0, k1, q0, q1, v0, v1 = _proj(x, wcat, bcat, bm=400)
    e0, e1 = eproj01
    part = _edge_sc(k0, k1, q0, q1, v0, v1, e0, e1, src_r, dst_r)
    return _combine(part, x, p['Ws'], p['b'], bm=400)


def kernel(x, edge_index, x_edge, params):
    src_r = edge_index[0]
    dst_r = edge_index[1]
    ep1 = _eproj(x_edge, params['conv1']['We'], params['conv1']['be'], bm=1000)
    ep2 = _eproj(x_edge, params['conv2']['We'], params['conv2']['be'], bm=1000)
    h = _layer(x, src_r, dst_r, ep1, params['conv1'])
    h = _layer(h, src_r, dst_r, ep2, params['conv2'])
    return h
